# baseline (speedup 1.0000x reference)
import functools

import jax
import jax.numpy as jnp
from jax import lax
from jax.experimental import pallas as pl
from jax.experimental.pallas import tpu as pltpu
from jax.experimental.pallas import tpu_sc as plsc

_CHUNK = 128
_NBUF = 8


@functools.cache
def _make_gather(num_chunks_total: int, dim: int):
    info = plsc.get_sparse_core_info()
    ncores, nsub = info.num_cores, info.num_subcores
    nw = ncores * nsub
    chunks_per_w = num_chunks_total // nw

    mesh = plsc.VectorSubcoreMesh(core_axis_name="c", subcore_axis_name="s")

    @functools.partial(
        pl.kernel,
        mesh=mesh,
        compiler_params=pltpu.CompilerParams(use_tc_tiling_on_sc=False),
        out_type=jax.ShapeDtypeStruct((num_chunks_total * _CHUNK, dim), jnp.float32),
        scratch_types=[
            pltpu.VMEM((chunks_per_w, _CHUNK), jnp.int32),
            pltpu.VMEM((_NBUF, _CHUNK, dim), jnp.float32),
        ]
        + [pltpu.SemaphoreType.DMA] * (2 * _NBUF),
    )
    def emb(idx_hbm, table_hbm, out_hbm, idx_v, rows_v, *sems):
        gsems, osems = sems[:_NBUF], sems[_NBUF:]
        wid = lax.axis_index("s") * ncores + lax.axis_index("c")
        chunk0 = wid * chunks_per_w
        pltpu.sync_copy(idx_hbm.at[pl.ds(chunk0, chunks_per_w)], idx_v)

        def gather(j, b):
            pltpu.async_copy(table_hbm.at[idx_v.at[j]], rows_v.at[b], gsems[b])

        def gather_wait(b):
            pltpu.make_async_copy(
                table_hbm.at[pl.ds(0, _CHUNK)], rows_v.at[b], gsems[b]
            ).wait()

        def store(j, b):
            pltpu.async_copy(
                rows_v.at[b],
                out_hbm.at[pl.ds((chunk0 + j) * _CHUNK, _CHUNK)],
                osems[b],
            )

        def store_wait(b):
            pltpu.make_async_copy(
                rows_v.at[b], out_hbm.at[pl.ds(0, _CHUNK)], osems[b]
            ).wait()

        for b in range(_NBUF):
            gather(b, b)

        gather_wait(0)
        store(0, 0)

        @pl.loop(1, chunks_per_w - _NBUF + 1, step=_NBUF)
        def _(g):
            for u in range(_NBUF):
                j = g + u
                bp = (u + _NBUF) % _NBUF
                b = (u + 1) % _NBUF
                store_wait(bp)
                gather(j + _NBUF - 1, bp)
                gather_wait(b)
                store(j, b)

        for j in range(chunks_per_w - _NBUF + 1, chunks_per_w):
            b = j % _NBUF
            gather_wait(b)
            store(j, b)
        for j in range(chunks_per_w - _NBUF, chunks_per_w):
            store_wait(j % _NBUF)

    return emb


def kernel(token_ids, weight):
    shape = token_ids.shape
    dim = weight.shape[1]
    flat = token_ids.reshape(-1).astype(jnp.int32)
    n = flat.shape[0]
    block = _CHUNK * 32
    pad = (-n) % block
    if pad:
        flat = jnp.concatenate([flat, jnp.zeros((pad,), jnp.int32)])
    num_chunks = (n + pad) // _CHUNK
    idx2d = flat.reshape(num_chunks, _CHUNK)
    out = _make_gather(num_chunks, dim)(idx2d, weight)
    if pad:
        out = out[:n]
    return out.reshape(*shape, dim)
